# Initial kernel scaffold; baseline (speedup 1.0000x reference)
import jax
import jax.numpy as jnp
from jax.experimental import pallas as pl


def kernel(u, aux, W_emb, W_g, W_phi, W_theta):
    raise NotImplementedError("write your pallas kernel here")


# speedup vs baseline: 25.5961x; 25.5961x over previous
import jax
import jax.numpy as jnp
from jax.experimental import pallas as pl

_C, _AUXC, _H, _W = 256, 64, 64, 64
_EMB = 64
_WS = 7
_PS = 5
_K = 12
_N = _H * _W
_NWIN = _WS * _WS
_CHA = 1024
_CHB = 512
_NA = _N // _CHA
_NB = _N // _CHB


def _offsets(radius):
    return [(di, dj, di * _W + dj)
            for di in range(-radius, radius + 1)
            for dj in range(-radius, radius + 1)]


def _halo_specs(nchan, ch, nblk):
    return [
        pl.BlockSpec((nchan, ch), lambda k: (0, jnp.maximum(k - 1, 0))),
        pl.BlockSpec((nchan, ch), lambda k: (0, k)),
        pl.BlockSpec((nchan, ch), lambda k, n=nblk: (0, jnp.minimum(k + 1, n - 1))),
    ]


def _mask(pg, dj, o):
    wcol = pg & (_W - 1)
    return ((wcol + dj >= 0) & (wcol + dj < _W)
            & (pg + o >= 0) & (pg + o < _N))


def _conv_body(auxl_ref, auxc_ref, auxr_ref, Wr_ref, Wphi_ref, Wth_ref,
               phi_ref, th_ref):
    f32 = jnp.float32
    pg = (jax.lax.broadcasted_iota(jnp.int32, (1, _CHA), 1)
          + pl.program_id(0) * _CHA)
    loc = jnp.concatenate([auxl_ref[:], auxc_ref[:], auxr_ref[:]], axis=1)
    acc = jnp.zeros((_EMB, _CHA), f32)
    for i in range(_PS):
        for j in range(_PS):
            di, dj = i - _PS // 2, j - _PS // 2
            o = di * _W + dj
            m = _mask(pg, dj, o).astype(f32)
            sl = loc[:, _CHA + o:2 * _CHA + o]
            acc = acc + jnp.dot(Wr_ref[i * _PS + j], sl * m,
                                preferred_element_type=f32)
    phi_ref[:] = jnp.dot(Wphi_ref[:], acc, preferred_element_type=f32)
    th_ref[:] = jnp.dot(Wth_ref[:], acc, preferred_element_type=f32)


def _att_body(phi_ref, thl_ref, thc_ref, thr_ref, wq_ref):
    f32 = jnp.float32
    pg = (jax.lax.broadcasted_iota(jnp.int32, (1, _CHA), 1)
          + pl.program_id(0) * _CHA)
    loc = jnp.concatenate([thl_ref[:], thc_ref[:], thr_ref[:]], axis=1)
    phi = phi_ref[:]
    atts, masks = [], []
    for di, dj, o in _offsets(_WS // 2):
        sl = loc[:, _CHA + o:2 * _CHA + o]
        a = jnp.sum(phi * sl, axis=0, keepdims=True)
        mj = _mask(pg, dj, o)
        atts.append(jnp.where(mj, a, 0.0))
        masks.append(mj.astype(f32))
    att = jnp.concatenate(atts, axis=0)

    m0 = jnp.max(att, axis=0, keepdims=True)
    t = jnp.full((1, _CHA), jnp.inf, f32)
    cnt = jnp.zeros((1, _CHA), f32)
    for _ in range(_K):
        active = cnt < _K
        nv = jnp.max(jnp.where(att < t, att, -jnp.inf), axis=0, keepdims=True)
        c = jnp.sum((att == nv).astype(f32), axis=0, keepdims=True)
        t = jnp.where(active, nv, t)
        cnt = jnp.where(active, cnt + c, cnt)
    ngt = jnp.sum((att > t).astype(f32), axis=0, keepdims=True)
    neq = _K - ngt
    et = jnp.exp(t - m0)

    pc = jnp.zeros((1, _CHA), f32)
    ws = []
    for o in range(_NWIN):
        a = atts[o]
        eq = (a == t).astype(f32)
        wv = jnp.where(a > t, jnp.exp(a - m0), 0.0) \
            + jnp.where((eq > 0) & (pc < neq), et, 0.0)
        pc = pc + eq
        ws.append(wv)
    denom = ws[0]
    for o in range(1, _NWIN):
        denom = denom + ws[o]
    rden = 1.0 / denom
    wq_ref[:] = jnp.concatenate(
        [ws[o] * masks[o] for o in range(_NWIN)], axis=0) * rden


def _g_body(u_ref, Wg_ref, g_ref):
    g_ref[:] = jnp.dot(Wg_ref[:], u_ref[:], preferred_element_type=jnp.float32)


def _comb_body(wq_ref, gl_ref, gc_ref, gr_ref, out_ref):
    f32 = jnp.float32
    pg = (jax.lax.broadcasted_iota(jnp.int32, (1, _CHB), 1)
          + pl.program_id(0) * _CHB)
    loc = jnp.concatenate([gl_ref[:], gc_ref[:], gr_ref[:]], axis=1)
    acc = jnp.zeros((_C, _CHB), f32)
    idx = 0
    for di, dj, o in _offsets(_WS // 2):
        wrow = wq_ref[idx:idx + 1, :]
        acc = acc + wrow * loc[:, _CHB + o:2 * _CHB + o]
        idx += 1
    out_ref[:] = acc


def kernel(u, aux, W_emb, W_g, W_phi, W_theta):
    b = u.shape[0]
    f32 = jnp.float32
    u2 = u.reshape(_C, _N)
    aux2 = aux.reshape(_AUXC, _N)
    Wr = W_emb.reshape(_EMB, _AUXC, _PS * _PS).transpose(2, 0, 1)

    full = lambda s: pl.BlockSpec(s, lambda k: (0,) * len(s))

    phi, th = pl.pallas_call(
        _conv_body,
        grid=(_NA,),
        in_specs=[*_halo_specs(_AUXC, _CHA, _NA),
                  full((_PS * _PS, _EMB, _AUXC)),
                  full((_EMB, _EMB)), full((_EMB, _EMB))],
        out_specs=[pl.BlockSpec((_EMB, _CHA), lambda k: (0, k))] * 2,
        out_shape=[jax.ShapeDtypeStruct((_EMB, _N), f32)] * 2,
    )(aux2, aux2, aux2, Wr, W_phi, W_theta)

    wq = pl.pallas_call(
        _att_body,
        grid=(_NA,),
        in_specs=[pl.BlockSpec((_EMB, _CHA), lambda k: (0, k)),
                  *_halo_specs(_EMB, _CHA, _NA)],
        out_specs=pl.BlockSpec((_NWIN, _CHA), lambda k: (0, k)),
        out_shape=jax.ShapeDtypeStruct((_NWIN, _N), f32),
    )(phi, th, th, th)

    g = pl.pallas_call(
        _g_body,
        grid=(_NA,),
        in_specs=[pl.BlockSpec((_C, _CHA), lambda k: (0, k)),
                  full((_C, _C))],
        out_specs=pl.BlockSpec((_C, _CHA), lambda k: (0, k)),
        out_shape=jax.ShapeDtypeStruct((_C, _N), f32),
    )(u2, W_g)

    out = pl.pallas_call(
        _comb_body,
        grid=(_NB,),
        in_specs=[pl.BlockSpec((_NWIN, _CHB), lambda k: (0, k)),
                  *_halo_specs(_C, _CHB, _NB)],
        out_specs=pl.BlockSpec((_C, _CHB), lambda k: (0, k)),
        out_shape=jax.ShapeDtypeStruct((_C, _N), f32),
    )(wq, g, g, g)

    return out.reshape(b, _C, _H, _W)
